# Initial kernel scaffold; baseline (speedup 1.0000x reference)
import jax
import jax.numpy as jnp
from jax.experimental import pallas as pl


def kernel(X, tables, var_table, W, b):
    raise NotImplementedError("write your pallas kernel here")


# speedup vs baseline: 2.3558x; 2.3558x over previous
import jax
import jax.numpy as jnp
from jax import lax
from jax.experimental import pallas as pl
from jax.experimental.pallas import tpu as pltpu
from jax.experimental.pallas import tpu_sc as plsc

B = 4096
F = 26
V = 100000
D = 16
L = 50

NC = 2
NS = 16
NW = NC * NS
EPW = B // NW
CH = 16
NCHUNK = EPW // CH
XROW = F + L


def _body(x_hbm, tab_hbm, var_hbm, w_hbm, b_hbm, out_hbm,
          xbuf, wbuf, bbuf, v0buf, sidx, qidx, srows, qrows,
          acc_a, acc_q, n0buf, outbuf, sem_s, sem_q):
    wid = lax.axis_index("s") * NC + lax.axis_index("c")
    base = pl.multiple_of(wid * EPW, EPW)

    pltpu.sync_copy(x_hbm.at[pl.ds(pl.multiple_of(wid * (EPW * XROW), 8),
                                   EPW * XROW)], xbuf)
    pltpu.sync_copy(w_hbm, wbuf)
    pltpu.sync_copy(b_hbm, bbuf)
    pltpu.sync_copy(var_hbm.at[pl.ds(0, 1)], v0buf)

    lanes = lax.iota(jnp.int32, 16)
    bvec = bbuf[...]
    wvar = wbuf[F]
    v0 = v0buf[0]

    def chunk_body(c, _):
        e0 = pl.multiple_of(c * CH, CH)
        xoff = (e0 + lanes) * XROW

        def sfill(f, _):
            v = plsc.load_gather(xbuf, [xoff + f])
            sidx[pl.ds(pl.multiple_of(f * CH, CH), CH)] = v + f * V
            return _
        lax.fori_loop(0, F, sfill, None)

        def qfill(l, n0):
            v = plsc.load_gather(xbuf, [xoff + (F + l)])
            qidx[pl.ds(pl.multiple_of(l * CH, CH), CH)] = v
            return n0 + jnp.where(v == 0, 1.0, 0.0)
        n0 = lax.fori_loop(0, L, qfill, jnp.zeros((16,), jnp.float32))
        n0buf[pl.ds(e0, CH)] = n0

        cp_s = pltpu.make_async_copy(tab_hbm.at[sidx], srows, sem_s)
        cp_q = pltpu.make_async_copy(var_hbm.at[qidx], qrows, sem_q)
        cp_s.start()
        cp_q.start()
        cp_s.wait()
        cp_q.wait()

        def elem_body(e, _):
            acc_s = jnp.zeros((16,), jnp.float32)
            for f in range(F):
                acc_s = acc_s + srows[f * CH + e] * wbuf[f]
            acc = jnp.zeros((16,), jnp.float32)
            for l in range(L):
                acc = acc + qrows[l * CH + e]
            n0_e = plsc.load_gather(n0buf, [jnp.full((16,), e0 + e, jnp.int32)])
            acc_a[pl.ds(pl.multiple_of(e * D, D), D)] = acc_s
            acc_q[pl.ds(pl.multiple_of(e * D, D), D)] = (acc - n0_e * v0) * wvar
            return _
        lax.fori_loop(0, CH, elem_body, None)

        def red_body(d, carry):
            dsp, dq = carry
            col = lanes * D + d
            dsp = dsp + plsc.load_gather(acc_a, [col])
            dq = dq + plsc.load_gather(acc_q, [col])
            return dsp, dq
        dsp, dq = lax.fori_loop(
            0, D, red_body,
            (jnp.zeros((16,), jnp.float32), jnp.zeros((16,), jnp.float32)))

        cnt = jnp.float32(L) - n0
        logit = dsp + dq / (cnt + 1e-8) + bvec
        sig = 1.0 / (1.0 + jnp.exp(-logit))
        outbuf[pl.ds(e0, CH)] = sig
        return _

    lax.fori_loop(0, NCHUNK, chunk_body, None)
    pltpu.sync_copy(outbuf, out_hbm.at[pl.ds(base, EPW)])


@jax.jit
def _run(x_flat, tab_flat, var_table, w2, b16):
    mesh = plsc.VectorSubcoreMesh(core_axis_name="c", subcore_axis_name="s")
    kfn = pl.kernel(
        _body,
        out_type=jax.ShapeDtypeStruct((B,), jnp.float32),
        mesh=mesh,
        compiler_params=pltpu.CompilerParams(
            needs_layout_passes=False, use_tc_tiling_on_sc=False),
        scratch_types=[
            pltpu.VMEM((EPW * XROW,), jnp.int32),
            pltpu.VMEM((F + 1, D), jnp.float32),
            pltpu.VMEM((16,), jnp.float32),
            pltpu.VMEM((1, D), jnp.float32),
            pltpu.VMEM((F * CH,), jnp.int32),
            pltpu.VMEM((L * CH,), jnp.int32),
            pltpu.VMEM((F * CH, D), jnp.float32),
            pltpu.VMEM((L * CH, D), jnp.float32),
            pltpu.VMEM((CH * D,), jnp.float32),
            pltpu.VMEM((CH * D,), jnp.float32),
            pltpu.VMEM((EPW,), jnp.float32),
            pltpu.VMEM((EPW,), jnp.float32),
            pltpu.SemaphoreType.DMA,
            pltpu.SemaphoreType.DMA,
        ],
    )
    return kfn(x_flat, tab_flat, var_table, w2, b16)


def kernel(X, tables, var_table, W, b):
    x_flat = X.reshape(B * XROW)
    tab_flat = tables.reshape(F * V, D)
    w2 = W.reshape(F + 1, D)
    b16 = jnp.broadcast_to(b.astype(jnp.float32), (16,))
    out = _run(x_flat, tab_flat, var_table, w2, b16)
    return out.reshape(B, 1)


# speedup vs baseline: 3.3743x; 1.4324x over previous
import functools

import jax
import jax.numpy as jnp
from jax import lax
from jax.experimental import pallas as pl
from jax.experimental.pallas import tpu as pltpu
from jax.experimental.pallas import tpu_sc as plsc

B = 4096
F = 26
V = 100000
D = 16
L = 50

VP = 100352
VBLK = 2048
NVB = VP // VBLK
PROWS = F * VP // D

NC = 2
NS = 16
NW = NC * NS
EPW = B // NW
CH = 16
NCHUNK = EPW // CH
XROW = F + L


def _p_body(t_ref, w_ref, o_ref):
    t = t_ref[0]
    w = w_ref[0, 0]
    o_ref[...] = (t * w[:, None]).sum(axis=0).reshape(VBLK // 128, 128)


def _tc_p(tab_t, w2):
    return pl.pallas_call(
        _p_body,
        grid=(F, NVB),
        in_specs=[
            pl.BlockSpec((1, D, VBLK), lambda f, k: (f, 0, k)),
            pl.BlockSpec((1, 1, D), lambda f, k: (f, 0, 0)),
        ],
        out_specs=pl.BlockSpec((VBLK // 128, 128),
                               lambda f, k: (f * NVB + k, 0)),
        out_shape=jax.ShapeDtypeStruct((F * VP // 128, 128), jnp.float32),
    )(tab_t, w2)


def _q_body(t_ref, w_ref, o_ref):
    t = t_ref[...]
    w = w_ref[0, 0]
    o_ref[...] = (t * w[:, None]).sum(axis=0).reshape(VBLK // 128, 128)


def _tc_q(var_t, w2):
    return pl.pallas_call(
        _q_body,
        grid=(NVB,),
        in_specs=[
            pl.BlockSpec((D, VBLK), lambda k: (0, k)),
            pl.BlockSpec((1, 1, D), lambda k: (F, 0, 0)),
        ],
        out_specs=pl.BlockSpec((VBLK // 128, 128), lambda k: (k, 0)),
        out_shape=jax.ShapeDtypeStruct((VP // 128, 128), jnp.float32),
    )(var_t, w2)


def _sc_body(x_hbm, p_hbm, q_hbm, b_hbm, out_hbm,
             xbuf, qbuf, bbuf, sidx, srows, outbuf, sem_s):
    wid = lax.axis_index("s") * NC + lax.axis_index("c")
    base = pl.multiple_of(wid * EPW, EPW)

    pltpu.sync_copy(x_hbm.at[:, pl.ds(base, EPW)], xbuf)
    pltpu.sync_copy(q_hbm, qbuf)
    pltpu.sync_copy(b_hbm, bbuf)

    lanes = lax.iota(jnp.int32, 16)
    bvec = bbuf[...]

    def chunk_body(c, _):
        e0 = pl.multiple_of(c * CH, CH)
        elane = e0 + lanes

        def sfill(f, _):
            xv = plsc.load_gather(xbuf, [jnp.full((16,), f, jnp.int32), elane])
            sidx[pl.ds(pl.multiple_of(f * CH, CH), CH)] = \
                f * (VP // D) + lax.shift_right_logical(xv, 4)
            return _
        lax.fori_loop(0, F, sfill, None)

        cp = pltpu.make_async_copy(p_hbm.at[sidx], srows, sem_s)
        cp.start()

        def qstep(l, carry):
            sq, n0 = carry
            xv = plsc.load_gather(
                xbuf, [jnp.full((16,), F + l, jnp.int32), elane])
            val = plsc.load_gather(qbuf, [xv])
            live = xv != 0
            sq = sq + jnp.where(live, val, 0.0)
            n0 = n0 + jnp.where(live, 0.0, 1.0)
            return sq, n0
        sq, n0 = lax.fori_loop(
            0, L, qstep,
            (jnp.zeros((16,), jnp.float32), jnp.zeros((16,), jnp.float32)))

        cp.wait()

        def pstep(f, sp):
            xv = plsc.load_gather(xbuf, [jnp.full((16,), f, jnp.int32), elane])
            val = plsc.load_gather(
                srows, [f * CH + lanes, jnp.bitwise_and(xv, D - 1)])
            return sp + val
        sp = lax.fori_loop(0, F, pstep, jnp.zeros((16,), jnp.float32))

        cnt = jnp.float32(L) - n0
        logit = sp + sq / (cnt + 1e-8) + bvec
        outbuf[pl.ds(e0, CH)] = 1.0 / (1.0 + jnp.exp(-logit))
        return _

    lax.fori_loop(0, NCHUNK, chunk_body, None)
    pltpu.sync_copy(outbuf, out_hbm.at[pl.ds(base, EPW)])


@jax.jit
def _run(x_t, tab_t, var_t, w2, b16):
    w3 = w2.reshape(F + 1, 1, D)
    p = _tc_p(tab_t, w3).reshape(PROWS, D)
    q = _tc_q(var_t, w3).reshape(VP)

    mesh = plsc.VectorSubcoreMesh(core_axis_name="c", subcore_axis_name="s")
    kfn = pl.kernel(
        _sc_body,
        out_type=jax.ShapeDtypeStruct((B,), jnp.float32),
        mesh=mesh,
        compiler_params=pltpu.CompilerParams(
            needs_layout_passes=False, use_tc_tiling_on_sc=False),
        scratch_types=[
            pltpu.VMEM((XROW, EPW), jnp.int32),
            pltpu.VMEM((VP,), jnp.float32),
            pltpu.VMEM((16,), jnp.float32),
            pltpu.VMEM((F * CH,), jnp.int32),
            pltpu.VMEM((F * CH, D), jnp.float32),
            pltpu.VMEM((EPW,), jnp.float32),
            pltpu.SemaphoreType.DMA,
        ],
    )
    return kfn(x_t, p, q, b16)


def kernel(X, tables, var_table, W, b):
    tab_t = jnp.transpose(tables, (0, 2, 1))
    var_t = var_table.T
    x_t = X.T
    w2 = W.reshape(F + 1, D)
    b16 = jnp.broadcast_to(b.astype(jnp.float32), (16,))
    out = _run(x_t, tab_t, var_t, w2, b16)
    return out.reshape(B, 1)


# speedup vs baseline: 20.3768x; 6.0388x over previous
import functools

import jax
import jax.numpy as jnp
from jax import lax
from jax.experimental import pallas as pl
from jax.experimental.pallas import tpu as pltpu
from jax.experimental.pallas import tpu_sc as plsc

B = 4096
F = 26
V = 100000
D = 16
L = 50

VP = 100352
VBLK = 50176
NVB = VP // VBLK
PROWS = F * VP // D

NC = 2
NS = 16
NW = NC * NS
EPW = B // NW
CH = 16
NCHUNK = EPW // CH
XROW = F + L


def _p_body(t_ref, w_ref, o_ref):
    t = t_ref[0]
    w = w_ref[0, 0]
    o_ref[...] = (t * w[:, None]).sum(axis=0).reshape(VBLK // 128, 128)


def _tc_p(tab_t, w2):
    return pl.pallas_call(
        _p_body,
        grid=(F, NVB),
        in_specs=[
            pl.BlockSpec((1, D, VBLK), lambda f, k: (f, 0, k)),
            pl.BlockSpec((1, 1, D), lambda f, k: (f, 0, 0)),
        ],
        out_specs=pl.BlockSpec((VBLK // 128, 128),
                               lambda f, k: (f * NVB + k, 0)),
        out_shape=jax.ShapeDtypeStruct((F * VP // 128, 128), jnp.float32),
    )(tab_t, w2)


def _q_body(t_ref, w_ref, o_ref):
    t = t_ref[...]
    w = w_ref[0, 0]
    o_ref[...] = (t * w[:, None]).sum(axis=0).reshape(VBLK // 128, 128)


def _tc_q(var_t, w2):
    return pl.pallas_call(
        _q_body,
        grid=(NVB,),
        in_specs=[
            pl.BlockSpec((D, VBLK), lambda k: (0, k)),
            pl.BlockSpec((1, 1, D), lambda k: (F, 0, 0)),
        ],
        out_specs=pl.BlockSpec((VBLK // 128, 128), lambda k: (k, 0)),
        out_shape=jax.ShapeDtypeStruct((VP // 128, 128), jnp.float32),
    )(var_t, w2)


def _sc_body(x_hbm, p_hbm, q_hbm, b_hbm, out_hbm,
             xbuf, qbuf, bbuf, sidx, srows, outbuf, sem_s):
    wid = lax.axis_index("s") * NC + lax.axis_index("c")
    base = pl.multiple_of(wid * EPW, EPW)

    pltpu.sync_copy(x_hbm.at[:, pl.ds(base, EPW)], xbuf)
    pltpu.sync_copy(q_hbm, qbuf)
    pltpu.sync_copy(b_hbm, bbuf)

    lanes = lax.iota(jnp.int32, 16)
    bvec = bbuf[...]

    def chunk_body(c, _):
        e0 = pl.multiple_of(c * CH, CH)
        elane = e0 + lanes

        def sfill(f, _):
            xv = plsc.load_gather(xbuf, [jnp.full((16,), f, jnp.int32), elane])
            sidx[pl.ds(pl.multiple_of(f * CH, CH), CH)] = \
                f * (VP // D) + lax.shift_right_logical(xv, 4)
            return _
        lax.fori_loop(0, F, sfill, None)

        cp = pltpu.make_async_copy(p_hbm.at[sidx], srows, sem_s)
        cp.start()

        def qstep(l, carry):
            sq, n0 = carry
            xv = plsc.load_gather(
                xbuf, [jnp.full((16,), F + l, jnp.int32), elane])
            val = plsc.load_gather(qbuf, [xv])
            live = xv != 0
            sq = sq + jnp.where(live, val, 0.0)
            n0 = n0 + jnp.where(live, 0.0, 1.0)
            return sq, n0
        sq, n0 = lax.fori_loop(
            0, L, qstep,
            (jnp.zeros((16,), jnp.float32), jnp.zeros((16,), jnp.float32)))

        cp.wait()

        def pstep(f, sp):
            xv = plsc.load_gather(xbuf, [jnp.full((16,), f, jnp.int32), elane])
            val = plsc.load_gather(
                srows, [f * CH + lanes, jnp.bitwise_and(xv, D - 1)])
            return sp + val
        sp = lax.fori_loop(0, F, pstep, jnp.zeros((16,), jnp.float32))

        cnt = jnp.float32(L) - n0
        logit = sp + sq / (cnt + 1e-8) + bvec
        outbuf[pl.ds(e0, CH)] = 1.0 / (1.0 + jnp.exp(-logit))
        return _

    lax.fori_loop(0, NCHUNK, chunk_body, None)
    pltpu.sync_copy(outbuf, out_hbm.at[pl.ds(base, EPW)])


@jax.jit
def _run(x_t, tab_t, var_t, w2, b16):
    w3 = w2.reshape(F + 1, 1, D)
    p = _tc_p(tab_t, w3).reshape(PROWS, D)
    q = _tc_q(var_t, w3).reshape(VP)

    mesh = plsc.VectorSubcoreMesh(core_axis_name="c", subcore_axis_name="s")
    kfn = pl.kernel(
        _sc_body,
        out_type=jax.ShapeDtypeStruct((B,), jnp.float32),
        mesh=mesh,
        compiler_params=pltpu.CompilerParams(
            needs_layout_passes=False, use_tc_tiling_on_sc=False),
        scratch_types=[
            pltpu.VMEM((XROW, EPW), jnp.int32),
            pltpu.VMEM((VP,), jnp.float32),
            pltpu.VMEM((16,), jnp.float32),
            pltpu.VMEM((F * CH,), jnp.int32),
            pltpu.VMEM((F * CH, D), jnp.float32),
            pltpu.VMEM((EPW,), jnp.float32),
            pltpu.SemaphoreType.DMA,
        ],
    )
    return kfn(x_t, p, q, b16)


def kernel(X, tables, var_table, W, b):
    tab_t = jnp.transpose(tables, (0, 2, 1))
    var_t = var_table.T
    x_t = X.T
    w2 = W.reshape(F + 1, D)
    b16 = jnp.broadcast_to(b.astype(jnp.float32), (16,))
    out = _run(x_t, tab_t, var_t, w2, b16)
    return out.reshape(B, 1)


# speedup vs baseline: 21.8818x; 1.0739x over previous
import functools

import jax
import jax.numpy as jnp
from jax import lax
from jax.experimental import pallas as pl
from jax.experimental.pallas import tpu as pltpu
from jax.experimental.pallas import tpu_sc as plsc

B = 4096
F = 26
V = 100000
D = 16
L = 50

VP = 100352
VBLK = 50176
NVB = VP // VBLK
PROWS = F * VP // D

NC = 2
NS = 16
NW = NC * NS
EPW = B // NW
CH = 16
NCHUNK = EPW // CH
XROW = F + L


def _p_body(t_ref, w_ref, o_ref):
    t = t_ref[0]
    w = w_ref[0]
    o_ref[...] = jnp.dot(w, t, preferred_element_type=jnp.float32
                         ).reshape(VBLK // 128, 128)


def _tc_p(tab_t, w2):
    return pl.pallas_call(
        _p_body,
        grid=(F, NVB),
        in_specs=[
            pl.BlockSpec((1, D, VBLK), lambda f, k: (f, 0, k)),
            pl.BlockSpec((1, 1, D), lambda f, k: (f, 0, 0)),
        ],
        out_specs=pl.BlockSpec((VBLK // 128, 128),
                               lambda f, k: (f * NVB + k, 0)),
        out_shape=jax.ShapeDtypeStruct((F * VP // 128, 128), jnp.float32),
    )(tab_t, w2)


def _q_body(t_ref, w_ref, o_ref):
    t = t_ref[...]
    w = w_ref[0]
    o_ref[...] = jnp.dot(w, t, preferred_element_type=jnp.float32
                         ).reshape(VBLK // 128, 128)


def _tc_q(var_t, w2):
    return pl.pallas_call(
        _q_body,
        grid=(NVB,),
        in_specs=[
            pl.BlockSpec((D, VBLK), lambda k: (0, k)),
            pl.BlockSpec((1, 1, D), lambda k: (F, 0, 0)),
        ],
        out_specs=pl.BlockSpec((VBLK // 128, 128), lambda k: (k, 0)),
        out_shape=jax.ShapeDtypeStruct((VP // 128, 128), jnp.float32),
    )(var_t, w2)


def _sc_body(x_hbm, p_hbm, q_hbm, b_hbm, out_hbm,
             xbuf, qbuf, bbuf, sidx, srows, outbuf, sem_s):
    wid = lax.axis_index("s") * NC + lax.axis_index("c")
    base = pl.multiple_of(wid * EPW, EPW)

    pltpu.sync_copy(x_hbm.at[:, pl.ds(base, EPW)], xbuf)
    pltpu.sync_copy(q_hbm, qbuf)
    pltpu.sync_copy(b_hbm, bbuf)

    lanes = lax.iota(jnp.int32, 16)
    bvec = bbuf[...]

    def chunk_body(c, _):
        e0 = pl.multiple_of(c * CH, CH)
        elane = e0 + lanes

        def sfill(f, _):
            xv = plsc.load_gather(xbuf, [jnp.full((16,), f, jnp.int32), elane])
            sidx[pl.ds(pl.multiple_of(f * CH, CH), CH)] = \
                f * (VP // D) + lax.shift_right_logical(xv, 4)
            return _
        lax.fori_loop(0, F, sfill, None)

        cp = pltpu.make_async_copy(p_hbm.at[sidx], srows, sem_s)
        cp.start()

        def qstep(l, carry):
            sq, n0 = carry
            xv = plsc.load_gather(
                xbuf, [jnp.full((16,), F + l, jnp.int32), elane])
            val = plsc.load_gather(qbuf, [xv])
            live = xv != 0
            sq = sq + jnp.where(live, val, 0.0)
            n0 = n0 + jnp.where(live, 0.0, 1.0)
            return sq, n0
        sq, n0 = lax.fori_loop(
            0, L, qstep,
            (jnp.zeros((16,), jnp.float32), jnp.zeros((16,), jnp.float32)))

        cp.wait()

        def pstep(f, sp):
            xv = plsc.load_gather(xbuf, [jnp.full((16,), f, jnp.int32), elane])
            val = plsc.load_gather(
                srows, [f * CH + lanes, jnp.bitwise_and(xv, D - 1)])
            return sp + val
        sp = lax.fori_loop(0, F, pstep, jnp.zeros((16,), jnp.float32))

        cnt = jnp.float32(L) - n0
        logit = sp + sq / (cnt + 1e-8) + bvec
        outbuf[pl.ds(e0, CH)] = 1.0 / (1.0 + jnp.exp(-logit))
        return _

    lax.fori_loop(0, NCHUNK, chunk_body, None)
    pltpu.sync_copy(outbuf, out_hbm.at[pl.ds(base, EPW)])


@jax.jit
def _run(x_t, tab_t, var_t, w2, b16):
    w3 = w2.reshape(F + 1, 1, D)
    p = _tc_p(tab_t, w3).reshape(PROWS, D)
    q = _tc_q(var_t, w3).reshape(VP)

    mesh = plsc.VectorSubcoreMesh(core_axis_name="c", subcore_axis_name="s")
    kfn = pl.kernel(
        _sc_body,
        out_type=jax.ShapeDtypeStruct((B,), jnp.float32),
        mesh=mesh,
        compiler_params=pltpu.CompilerParams(
            needs_layout_passes=False, use_tc_tiling_on_sc=False),
        scratch_types=[
            pltpu.VMEM((XROW, EPW), jnp.int32),
            pltpu.VMEM((VP,), jnp.float32),
            pltpu.VMEM((16,), jnp.float32),
            pltpu.VMEM((F * CH,), jnp.int32),
            pltpu.VMEM((F * CH, D), jnp.float32),
            pltpu.VMEM((EPW,), jnp.float32),
            pltpu.SemaphoreType.DMA,
        ],
    )
    return kfn(x_t, p, q, b16)


def kernel(X, tables, var_table, W, b):
    tab_t = jnp.transpose(tables, (0, 2, 1))
    var_t = var_table.T
    x_t = X.T
    w2 = W.reshape(F + 1, D)
    b16 = jnp.broadcast_to(b.astype(jnp.float32), (16,))
    out = _run(x_t, tab_t, var_t, w2, b16)
    return out.reshape(B, 1)
